# baseline (speedup 1.0000x reference)
import functools

import jax
import jax.numpy as jnp
from jax import lax
from jax.experimental import pallas as pl
from jax.experimental.pallas import tpu as pltpu
from jax.experimental.pallas import tpu_sc as plsc

_B, _C, _NUM_BANDS = 32, 256, 64
_NC, _NS = 2, 16
_GATHER_WORKERS = 4
_ROWS_PER_W = _B // _GATHER_WORKERS


def _sc_gather_body(table_hbm, idx_hbm, gamma_hbm, beta_hbm, idx_v, rows_v, sem):
    wid = lax.axis_index("s") * _NC + lax.axis_index("c")

    @pl.when(wid < _GATHER_WORKERS)
    def _():
        base = wid * _ROWS_PER_W
        pltpu.sync_copy(idx_hbm.at[pl.ds(base, _ROWS_PER_W)], idx_v)
        pltpu.async_copy(table_hbm.at[idx_v], rows_v, sem).wait()
        pltpu.sync_copy(rows_v.at[:, pl.ds(0, _C)], gamma_hbm.at[pl.ds(base, _ROWS_PER_W)])
        pltpu.sync_copy(rows_v.at[:, pl.ds(_C, _C)], beta_hbm.at[pl.ds(base, _ROWS_PER_W)])


@jax.jit
def _sc_gather(embed_weight, idx):
    mesh = plsc.VectorSubcoreMesh(core_axis_name="c", subcore_axis_name="s")
    return pl.kernel(
        _sc_gather_body,
        out_type=(
            jax.ShapeDtypeStruct((_B, _C), jnp.float32),
            jax.ShapeDtypeStruct((_B, _C), jnp.float32),
        ),
        mesh=mesh,
        scratch_types=[
            pltpu.VMEM((_ROWS_PER_W,), jnp.int32),
            pltpu.VMEM((_ROWS_PER_W, 2 * _C), jnp.float32),
            pltpu.SemaphoreType.DMA,
        ],
    )(embed_weight, idx)


def _film_body(gamma_ref, beta_ref, x_ref, o_ref):
    g = (1.0 + gamma_ref[0, 0]).reshape(-1, 1, 1)
    b = beta_ref[0, 0].reshape(-1, 1, 1)
    o_ref[0] = x_ref[0] * g + b


def _film(gamma, beta, x, c_blk):
    B, C, H, W = x.shape
    grid = (B, C // c_blk)
    gamma = gamma.reshape(B, 1, C)
    beta = beta.reshape(B, 1, C)
    gb_spec = pl.BlockSpec((1, 1, c_blk), lambda b, c: (b, 0, c))
    x_spec = pl.BlockSpec((1, c_blk, H, W), lambda b, c: (b, c, 0, 0))
    return pl.pallas_call(
        _film_body,
        grid=grid,
        in_specs=[gb_spec, gb_spec, x_spec],
        out_specs=x_spec,
        out_shape=jax.ShapeDtypeStruct(x.shape, x.dtype),
        compiler_params=pltpu.CompilerParams(
            dimension_semantics=("parallel", "parallel"),
        ),
    )(gamma, beta, x)


def kernel(x, band_idx, embed_weight):
    idx = band_idx.astype(jnp.int32)
    gamma, beta = _sc_gather(embed_weight, idx)
    return _film(gamma, beta, x, c_blk=128)


# speedup vs baseline: 5.3773x; 5.3773x over previous
import functools

import jax
import jax.numpy as jnp
from jax import lax
from jax.experimental import pallas as pl
from jax.experimental.pallas import tpu as pltpu
from jax.experimental.pallas import tpu_sc as plsc

_B, _C, _NUM_BANDS = 32, 256, 64
_NC, _NS = 2, 16
_GATHER_WORKERS = 4
_ROWS_PER_W = _B // _GATHER_WORKERS


def _sc_gather_body(table_hbm, idx_hbm, gamma_hbm, beta_hbm, idx_v, rows_v, sem):
    wid = lax.axis_index("s") * _NC + lax.axis_index("c")

    @pl.when(wid < _GATHER_WORKERS)
    def _():
        base = wid * _ROWS_PER_W
        pltpu.sync_copy(idx_hbm.at[pl.ds(base, _ROWS_PER_W)], idx_v)
        pltpu.async_copy(table_hbm.at[idx_v], rows_v, sem).wait()
        pltpu.sync_copy(rows_v.at[:, pl.ds(0, _C)], gamma_hbm.at[pl.ds(base, _ROWS_PER_W)])
        pltpu.sync_copy(rows_v.at[:, pl.ds(_C, _C)], beta_hbm.at[pl.ds(base, _ROWS_PER_W)])


@jax.jit
def _sc_gather(embed_weight, idx):
    mesh = plsc.VectorSubcoreMesh(core_axis_name="c", subcore_axis_name="s")
    return pl.kernel(
        _sc_gather_body,
        out_type=(
            jax.ShapeDtypeStruct((_B, _C), jnp.float32),
            jax.ShapeDtypeStruct((_B, _C), jnp.float32),
        ),
        mesh=mesh,
        scratch_types=[
            pltpu.VMEM((_ROWS_PER_W,), jnp.int32),
            pltpu.VMEM((_ROWS_PER_W, 2 * _C), jnp.float32),
            pltpu.SemaphoreType.DMA,
        ],
    )(embed_weight, idx)


def _film_body(gamma_ref, beta_ref, x_ref, o_ref):
    g = 1.0 + gamma_ref[0, 0]
    b = beta_ref[0, 0]
    o_ref[0] = x_ref[0] * g + b


def _film(gamma, beta, xt, h_blk):
    B, H, W, C = xt.shape
    grid = (B, H // h_blk)
    gamma = gamma.reshape(B, 1, C)
    beta = beta.reshape(B, 1, C)
    gb_spec = pl.BlockSpec((1, 1, C), lambda b, h: (b, 0, 0))
    x_spec = pl.BlockSpec((1, h_blk, W, C), lambda b, h: (b, h, 0, 0))
    return pl.pallas_call(
        _film_body,
        grid=grid,
        in_specs=[gb_spec, gb_spec, x_spec],
        out_specs=x_spec,
        out_shape=jax.ShapeDtypeStruct(xt.shape, xt.dtype),
        compiler_params=pltpu.CompilerParams(
            dimension_semantics=("parallel", "parallel"),
        ),
    )(gamma, beta, xt)


def kernel(x, band_idx, embed_weight):
    idx = band_idx.astype(jnp.int32)
    gamma, beta = _sc_gather(embed_weight, idx)
    xt = jnp.transpose(x, (0, 2, 3, 1))
    out_t = _film(gamma, beta, xt, h_blk=32)
    return jnp.transpose(out_t, (0, 3, 1, 2))


# speedup vs baseline: 5.7864x; 1.0761x over previous
import functools

import jax
import jax.numpy as jnp
from jax import lax
from jax.experimental import pallas as pl
from jax.experimental.pallas import tpu as pltpu
from jax.experimental.pallas import tpu_sc as plsc

_B, _C, _NUM_BANDS = 32, 256, 64
_NC, _NS = 2, 16
_GATHER_WORKERS = 4
_ROWS_PER_W = _B // _GATHER_WORKERS


def _sc_gather_body(table_hbm, idx_hbm, gamma_hbm, beta_hbm, idx_v, rows_v, sem):
    wid = lax.axis_index("s") * _NC + lax.axis_index("c")

    @pl.when(wid < _GATHER_WORKERS)
    def _():
        base = wid * _ROWS_PER_W
        pltpu.sync_copy(idx_hbm.at[pl.ds(base, _ROWS_PER_W)], idx_v)
        pltpu.async_copy(table_hbm.at[idx_v], rows_v, sem).wait()
        pltpu.sync_copy(rows_v.at[:, pl.ds(0, _C)], gamma_hbm.at[pl.ds(base, _ROWS_PER_W)])
        pltpu.sync_copy(rows_v.at[:, pl.ds(_C, _C)], beta_hbm.at[pl.ds(base, _ROWS_PER_W)])


@jax.jit
def _sc_gather(embed_weight, idx):
    mesh = plsc.VectorSubcoreMesh(core_axis_name="c", subcore_axis_name="s")
    return pl.kernel(
        _sc_gather_body,
        out_type=(
            jax.ShapeDtypeStruct((_B, _C), jnp.float32),
            jax.ShapeDtypeStruct((_B, _C), jnp.float32),
        ),
        mesh=mesh,
        scratch_types=[
            pltpu.VMEM((_ROWS_PER_W,), jnp.int32),
            pltpu.VMEM((_ROWS_PER_W, 2 * _C), jnp.float32),
            pltpu.SemaphoreType.DMA,
        ],
    )(embed_weight, idx)


def _film_body(gamma_ref, beta_ref, x_ref, o_ref):
    g = 1.0 + gamma_ref[0, 0]
    b = beta_ref[0, 0]
    o_ref[0] = x_ref[0] * g + b


def _film(gamma, beta, xt, h_blk):
    B, H, W, C = xt.shape
    grid = (B, H // h_blk)
    gamma = gamma.reshape(B, 1, C)
    beta = beta.reshape(B, 1, C)
    gb_spec = pl.BlockSpec((1, 1, C), lambda b, h: (b, 0, 0))
    x_spec = pl.BlockSpec((1, h_blk, W, C), lambda b, h: (b, h, 0, 0))
    return pl.pallas_call(
        _film_body,
        grid=grid,
        in_specs=[gb_spec, gb_spec, x_spec],
        out_specs=x_spec,
        out_shape=jax.ShapeDtypeStruct(xt.shape, xt.dtype),
        compiler_params=pltpu.CompilerParams(
            dimension_semantics=("parallel", "parallel"),
        ),
    )(gamma, beta, xt)


def kernel(x, band_idx, embed_weight):
    idx = band_idx.astype(jnp.int32)
    gamma, beta = _sc_gather(embed_weight, idx)
    xt = jnp.transpose(x, (0, 2, 3, 1))
    out_t = _film(gamma, beta, xt, h_blk=64)
    return jnp.transpose(out_t, (0, 3, 1, 2))


# speedup vs baseline: 5.8772x; 1.0157x over previous
import functools

import jax
import jax.numpy as jnp
from jax import lax
from jax.experimental import pallas as pl
from jax.experimental.pallas import tpu as pltpu
from jax.experimental.pallas import tpu_sc as plsc

_B, _C, _NUM_BANDS = 32, 256, 64
_NC, _NS = 2, 16
_GATHER_WORKERS = 4
_ROWS_PER_W = _B // _GATHER_WORKERS


def _sc_gather_body(table_hbm, idx_hbm, gamma_hbm, beta_hbm, idx_v, rows_v, sem):
    wid = lax.axis_index("s") * _NC + lax.axis_index("c")

    @pl.when(wid < _GATHER_WORKERS)
    def _():
        base = wid * _ROWS_PER_W
        pltpu.sync_copy(idx_hbm.at[pl.ds(base, _ROWS_PER_W)], idx_v)
        pltpu.async_copy(table_hbm.at[idx_v], rows_v, sem).wait()
        pltpu.sync_copy(rows_v.at[:, pl.ds(0, _C)], gamma_hbm.at[pl.ds(base, _ROWS_PER_W)])
        pltpu.sync_copy(rows_v.at[:, pl.ds(_C, _C)], beta_hbm.at[pl.ds(base, _ROWS_PER_W)])


@jax.jit
def _sc_gather(embed_weight, idx):
    mesh = plsc.VectorSubcoreMesh(core_axis_name="c", subcore_axis_name="s")
    return pl.kernel(
        _sc_gather_body,
        out_type=(
            jax.ShapeDtypeStruct((_B, _C), jnp.float32),
            jax.ShapeDtypeStruct((_B, _C), jnp.float32),
        ),
        mesh=mesh,
        scratch_types=[
            pltpu.VMEM((_ROWS_PER_W,), jnp.int32),
            pltpu.VMEM((_ROWS_PER_W, 2 * _C), jnp.float32),
            pltpu.SemaphoreType.DMA,
        ],
    )(embed_weight, idx)


def _film_body(gamma_ref, beta_ref, x_ref, o_ref):
    g = 1.0 + gamma_ref[...][:, :, None, :]
    b = beta_ref[...][:, :, None, :]
    o_ref[...] = x_ref[...] * g + b


def _film(gamma, beta, xt, b_blk, h_blk):
    B, H, W, C = xt.shape
    grid = (B // b_blk, H // h_blk)
    gamma = gamma.reshape(B, 1, C)
    beta = beta.reshape(B, 1, C)
    gb_spec = pl.BlockSpec((b_blk, 1, C), lambda b, h: (b, 0, 0))
    x_spec = pl.BlockSpec((b_blk, h_blk, W, C), lambda b, h: (b, h, 0, 0))
    return pl.pallas_call(
        _film_body,
        grid=grid,
        in_specs=[gb_spec, gb_spec, x_spec],
        out_specs=x_spec,
        out_shape=jax.ShapeDtypeStruct(xt.shape, xt.dtype),
        compiler_params=pltpu.CompilerParams(
            dimension_semantics=("parallel", "parallel"),
        ),
    )(gamma, beta, xt)


def kernel(x, band_idx, embed_weight):
    idx = band_idx.astype(jnp.int32)
    gamma, beta = _sc_gather(embed_weight, idx)
    xt = jnp.transpose(x, (0, 2, 3, 1))
    out_t = _film(gamma, beta, xt, b_blk=2, h_blk=64)
    return jnp.transpose(out_t, (0, 3, 1, 2))


# speedup vs baseline: 5.9988x; 1.0207x over previous
import functools

import jax
import jax.numpy as jnp
from jax import lax
from jax.experimental import pallas as pl
from jax.experimental.pallas import tpu as pltpu
from jax.experimental.pallas import tpu_sc as plsc

_B, _C, _NUM_BANDS = 32, 256, 64
_NC, _NS = 2, 16
_GATHER_WORKERS = 4
_ROWS_PER_W = _B // _GATHER_WORKERS


def _sc_gather_body(table_hbm, idx_hbm, gamma_hbm, beta_hbm, idx_v, rows_v, sem):
    wid = lax.axis_index("s") * _NC + lax.axis_index("c")

    @pl.when(wid < _GATHER_WORKERS)
    def _():
        base = wid * _ROWS_PER_W
        pltpu.sync_copy(idx_hbm.at[pl.ds(base, _ROWS_PER_W)], idx_v)
        pltpu.async_copy(table_hbm.at[idx_v], rows_v, sem).wait()
        pltpu.sync_copy(rows_v.at[:, pl.ds(0, _C)], gamma_hbm.at[pl.ds(base, _ROWS_PER_W)])
        pltpu.sync_copy(rows_v.at[:, pl.ds(_C, _C)], beta_hbm.at[pl.ds(base, _ROWS_PER_W)])


@jax.jit
def _sc_gather(embed_weight, idx):
    mesh = plsc.VectorSubcoreMesh(core_axis_name="c", subcore_axis_name="s")
    return pl.kernel(
        _sc_gather_body,
        out_type=(
            jax.ShapeDtypeStruct((_B, _C), jnp.float32),
            jax.ShapeDtypeStruct((_B, _C), jnp.float32),
        ),
        mesh=mesh,
        scratch_types=[
            pltpu.VMEM((_ROWS_PER_W,), jnp.int32),
            pltpu.VMEM((_ROWS_PER_W, 2 * _C), jnp.float32),
            pltpu.SemaphoreType.DMA,
        ],
    )(embed_weight, idx)


_NBUF = 4
_CHUNK = 2048


def _film_body(gamma_ref, beta_ref, x_hbm, o_hbm, xb, ob, insems, outsems):
    M = x_hbm.shape[0]
    nchunk = M // _CHUNK
    rows_per_b = 64 * 64

    def in_copy(i, slot):
        return pltpu.make_async_copy(
            x_hbm.at[pl.ds(i * _CHUNK, _CHUNK)], xb.at[slot], insems.at[slot]
        )

    def out_copy(i, slot):
        return pltpu.make_async_copy(
            ob.at[slot], o_hbm.at[pl.ds(i * _CHUNK, _CHUNK)], outsems.at[slot]
        )

    for s in range(_NBUF):
        in_copy(s, s).start()

    def step(i, carry):
        slot = lax.rem(i, _NBUF)
        in_copy(i, slot).wait()

        @pl.when(i >= _NBUF)
        def _():
            out_copy(i - _NBUF, slot).wait()

        b = i // (rows_per_b // _CHUNK)
        g = 1.0 + gamma_ref[pl.ds(b, 1), :]
        bt = beta_ref[pl.ds(b, 1), :]
        ob[slot] = xb[slot] * g + bt

        out_copy(i, slot).start()

        @pl.when(i + _NBUF < nchunk)
        def _():
            in_copy(i + _NBUF, slot).start()

        return carry

    lax.fori_loop(0, nchunk, step, 0)
    for k in range(_NBUF):
        i = nchunk - _NBUF + k
        out_copy(i, i % _NBUF).wait()


def _film(gamma, beta, x2d):
    M, C = x2d.shape
    return pl.pallas_call(
        _film_body,
        in_specs=[
            pl.BlockSpec(memory_space=pltpu.VMEM),
            pl.BlockSpec(memory_space=pltpu.VMEM),
            pl.BlockSpec(memory_space=pl.ANY),
        ],
        out_specs=pl.BlockSpec(memory_space=pl.ANY),
        out_shape=jax.ShapeDtypeStruct(x2d.shape, x2d.dtype),
        scratch_shapes=[
            pltpu.VMEM((_NBUF, _CHUNK, C), jnp.float32),
            pltpu.VMEM((_NBUF, _CHUNK, C), jnp.float32),
            pltpu.SemaphoreType.DMA((_NBUF,)),
            pltpu.SemaphoreType.DMA((_NBUF,)),
        ],
    )(gamma, beta, x2d)


def kernel(x, band_idx, embed_weight):
    B, C, H, W = x.shape
    idx = band_idx.astype(jnp.int32)
    gamma, beta = _sc_gather(embed_weight, idx)
    x2d = jnp.transpose(x, (0, 2, 3, 1)).reshape(B * H * W, C)
    out2d = _film(gamma, beta, x2d)
    return jnp.transpose(out2d.reshape(B, H, W, C), (0, 3, 1, 2))
